# baseline (speedup 1.0000x reference)
import functools

import jax
import jax.numpy as jnp
from jax import lax
from jax.experimental import pallas as pl
from jax.experimental.pallas import tpu as pltpu
from jax.experimental.pallas import tpu_sc as plsc

N = 10000
E = 320000
D = 128
NC = 2
NS = 16
NW = NC * NS
EPT = E // NW
NP = 10240
K = 250
NCHUNK = EPT // K
NH = NCHUNK // 2
NPT = N // NS

_mesh = plsc.VectorSubcoreMesh(core_axis_name="c", subcore_axis_name="s")
_CP = pltpu.CompilerParams(use_tc_tiling_on_sc=False)


_CPD = pltpu.CompilerParams(use_tc_tiling_on_sc=False,
                            needs_layout_passes=False)


NR = NP // 16


@functools.partial(
    pl.kernel,
    out_type=jax.ShapeDtypeStruct((NC, NS, NR // NS, 16), jnp.float32),
    mesh=_mesh,
    compiler_params=_CPD,
    scratch_types=[
        pltpu.VMEM((EPT,), jnp.int32),
        pltpu.VMEM((5, 128), jnp.int32),
        pltpu.VMEM((NR, 16), jnp.float32),
        pltpu.VMEM_SHARED((NR, 16), jnp.float32),
    ],
)
def _sc_degree(dst_hbm, iota_hbm, deg_out, dst_v, iota_v, hist, degbuf):
    c = lax.axis_index("c")
    s = lax.axis_index("s")
    rpt = NR // NS

    def _zero(i, _):
        for t in range(5):
            hist[5 * i + t] = jnp.zeros((16,), jnp.float32)
        return 0

    lax.fori_loop(0, NR // 5, _zero, 0)
    pltpu.sync_copy(hist.at[pl.ds(0, rpt)], degbuf.at[pl.ds(s * rpt, rpt)])
    pltpu.sync_copy(dst_hbm.at[c, s], dst_v)
    pltpu.sync_copy(iota_hbm, iota_v)

    ones = jnp.ones((16,), jnp.float32)

    def _accum(i, _):
        for t in range(5):
            idx = dst_v[pl.ds((5 * i + t) * 16, 16)]
            plsc.addupdate_scatter(
                hist,
                [lax.shift_right_logical(idx, 4),
                 lax.bitwise_and(idx, 15)],
                ones)
        return 0

    lax.fori_loop(0, EPT // 80, _accum, 0)
    plsc.subcore_barrier()
    for t in range(5):
        pltpu.sync_copy(hist.at[pl.ds(t * 128, 128)],
                        degbuf.at[iota_v.at[t]], add=True)
    plsc.subcore_barrier()
    pltpu.sync_copy(degbuf.at[pl.ds(s * rpt, rpt)], deg_out.at[c, s])


def _tc_matmul_body(x_ref, w_ref, xw_ref):
    xw_ref[...] = jnp.dot(x_ref[...], w_ref[...],
                          preferred_element_type=jnp.float32)


def _tc_matmul(x, W):
    R = 1000
    return pl.pallas_call(
        _tc_matmul_body,
        grid=(N // R,),
        in_specs=[
            pl.BlockSpec((R, D), lambda i: (i, 0)),
            pl.BlockSpec((D, D), lambda i: (0, 0)),
        ],
        out_specs=pl.BlockSpec((R, D), lambda i: (i, 0)),
        out_shape=jax.ShapeDtypeStruct((N, D), jnp.float32),
    )(x, W)


def _tc_scale_body(deg0_ref, deg1_ref, xw_ref, y_ref, dinv_ref):
    deg = deg0_ref[...] + deg1_ref[...] + 1.0
    dinv = lax.rsqrt(deg)
    y_ref[...] = xw_ref[...] * dinv
    dinv_ref[...] = dinv


def _tc_scale(deg, xw):
    R = 1000
    grid = N // R
    return pl.pallas_call(
        _tc_scale_body,
        grid=(grid,),
        in_specs=[
            pl.BlockSpec((R, 1), lambda i: (i, 0)),
            pl.BlockSpec((R, 1), lambda i: (i, 0)),
            pl.BlockSpec((R, D), lambda i: (i, 0)),
        ],
        out_specs=[
            pl.BlockSpec((R, D), lambda i: (i, 0)),
            pl.BlockSpec((R, 1), lambda i: (i, 0)),
        ],
        out_shape=[
            jax.ShapeDtypeStruct((N, D), jnp.float32),
            jax.ShapeDtypeStruct((N, 1), jnp.float32),
        ],
    )(deg[0].reshape(NP, 1), deg[1].reshape(NP, 1), xw)


@functools.partial(
    pl.kernel,
    out_type=jax.ShapeDtypeStruct((NC, NS, NPT, D), jnp.float32),
    mesh=_mesh,
    compiler_params=_CP,
    scratch_types=[
        pltpu.VMEM((NCHUNK, K), jnp.int32),
        pltpu.VMEM((NH, K), jnp.int32),
        pltpu.VMEM((K, D), jnp.float32),
        pltpu.VMEM_SHARED((N, D), jnp.float32),
        pltpu.SemaphoreType.DMA,
    ],
)
def _sc_edges(src_hbm, dst_hbm, y_hbm, agg_out,
              src_v, dst_v, rows_v, aggbuf, sem):
    c = lax.axis_index("c")
    s = lax.axis_index("s")

    def _zero(i, _):
        for t in range(D // 16):
            rows_v[i, pl.ds(t * 16, 16)] = jnp.zeros((16,), jnp.float32)
        return 0

    lax.fori_loop(0, K, _zero, 0)
    nfull = NPT // K
    rem = NPT % K
    for t in range(nfull):
        pltpu.sync_copy(rows_v, aggbuf.at[pl.ds(s * NPT + t * K, K)])
    if rem:
        pltpu.sync_copy(rows_v.at[pl.ds(0, rem)],
                        aggbuf.at[pl.ds(s * NPT + nfull * K, rem)])
    pltpu.sync_copy(src_hbm.at[c, s], src_v)
    pltpu.sync_copy(dst_hbm.at[c, s, pl.ds(0, NH)], dst_v)
    plsc.subcore_barrier()

    for h in range(2):
        if h == 1:
            pltpu.sync_copy(dst_hbm.at[c, s, pl.ds(NH, NH)], dst_v)

        def _accum(j, _):
            pltpu.async_copy(y_hbm.at[src_v.at[h * NH + j]], rows_v,
                             sem).wait()
            pltpu.sync_copy(rows_v, aggbuf.at[dst_v.at[j]], add=True)
            return 0

        lax.fori_loop(0, NH, _accum, 0)
    plsc.subcore_barrier()
    pltpu.sync_copy(aggbuf.at[pl.ds(s * NPT, NPT)], agg_out.at[c, s])


def _tc_final_body(agg_ref, y_ref, dinv_ref, b_ref, o_ref):
    total = agg_ref[0] + agg_ref[1] + y_ref[...]
    o_ref[...] = jnp.maximum(total * dinv_ref[...] + b_ref[...], 0.0)


def _tc_final(agg, y, dinv, b):
    R = 1000
    grid = N // R
    return pl.pallas_call(
        _tc_final_body,
        grid=(grid,),
        in_specs=[
            pl.BlockSpec((NC, R, D), lambda i: (0, i, 0)),
            pl.BlockSpec((R, D), lambda i: (i, 0)),
            pl.BlockSpec((R, 1), lambda i: (i, 0)),
            pl.BlockSpec((1, D), lambda i: (0, 0)),
        ],
        out_specs=pl.BlockSpec((R, D), lambda i: (i, 0)),
        out_shape=jax.ShapeDtypeStruct((N, D), jnp.float32),
    )(agg, y, dinv, b)


@jax.jit
def kernel(x, edge_index, W, b):
    src = edge_index[0].reshape(NC, NS, NCHUNK, K)
    dst = edge_index[1].reshape(NC, NS, NCHUNK, K)
    dstd = edge_index[1].reshape(NC, NS, EPT)
    iota_rows = jnp.arange(NR, dtype=jnp.int32).reshape(5, 128)
    xw = _tc_matmul(x, W)
    deg = _sc_degree(dstd, iota_rows).reshape(NC, NP)
    y, dinv = _tc_scale(deg, xw)
    agg = _sc_edges(src, dst, y).reshape(NC, N, D)
    return _tc_final(agg, y, dinv, b.reshape(1, D))


# speedup vs baseline: 1.0093x; 1.0093x over previous
import functools

import jax
import jax.numpy as jnp
from jax import lax
from jax.experimental import pallas as pl
from jax.experimental.pallas import tpu as pltpu
from jax.experimental.pallas import tpu_sc as plsc

N = 10000
E = 320000
D = 128
NC = 2
NS = 16
NW = NC * NS
EPT = E // NW
KD = 200
NCHUNKD = EPT // KD
K = 250
NCHUNK = EPT // K
NH = NCHUNK // 2
NPT = N // NS

_mesh = plsc.VectorSubcoreMesh(core_axis_name="c", subcore_axis_name="s")
_CP = pltpu.CompilerParams(use_tc_tiling_on_sc=False)


@functools.partial(
    pl.kernel,
    out_type=jax.ShapeDtypeStruct((NC, NS, NPT, 16), jnp.float32),
    mesh=_mesh,
    compiler_params=_CP,
    scratch_types=[
        pltpu.VMEM((NCHUNKD, KD), jnp.int32),
        pltpu.VMEM((KD, 16), jnp.float32),
        pltpu.VMEM((NPT, 16), jnp.float32),
        pltpu.VMEM_SHARED((N, 16), jnp.float32),
    ],
)
def _sc_degree(dst_hbm, ones_hbm, deg_out, dst_v, ones_v, zbuf, degbuf):
    c = lax.axis_index("c")
    s = lax.axis_index("s")

    def _zero(i, _):
        zbuf[i] = jnp.zeros((16,), jnp.float32)
        return 0

    lax.fori_loop(0, NPT, _zero, 0)
    pltpu.sync_copy(zbuf, degbuf.at[pl.ds(s * NPT, NPT)])
    pltpu.sync_copy(dst_hbm.at[c, s], dst_v)
    pltpu.sync_copy(ones_hbm, ones_v)
    plsc.subcore_barrier()

    def _accum(j, _):
        pltpu.sync_copy(ones_v, degbuf.at[dst_v.at[j]], add=True)
        return 0

    lax.fori_loop(0, NCHUNKD, _accum, 0)
    plsc.subcore_barrier()
    pltpu.sync_copy(degbuf.at[pl.ds(s * NPT, NPT)], deg_out.at[c, s])


def _tc_matmul_body(x_ref, w_ref, xw_ref):
    xw_ref[...] = jnp.dot(x_ref[...], w_ref[...],
                          preferred_element_type=jnp.float32)


def _tc_matmul(x, W):
    R = 1000
    return pl.pallas_call(
        _tc_matmul_body,
        grid=(N // R,),
        in_specs=[
            pl.BlockSpec((R, D), lambda i: (i, 0)),
            pl.BlockSpec((D, D), lambda i: (0, 0)),
        ],
        out_specs=pl.BlockSpec((R, D), lambda i: (i, 0)),
        out_shape=jax.ShapeDtypeStruct((N, D), jnp.float32),
    )(x, W)


def _tc_scale_body(deg_ref, xw_ref, y_ref, dinv_ref):
    deg = deg_ref[0, :, 0] + deg_ref[1, :, 0] + 1.0
    dinv = lax.rsqrt(deg)
    y_ref[...] = xw_ref[...] * dinv[:, None]
    dinv_ref[...] = dinv[:, None]


def _tc_scale(deg, xw):
    R = 1000
    grid = N // R
    return pl.pallas_call(
        _tc_scale_body,
        grid=(grid,),
        in_specs=[
            pl.BlockSpec((NC, R, 16), lambda i: (0, i, 0)),
            pl.BlockSpec((R, D), lambda i: (i, 0)),
        ],
        out_specs=[
            pl.BlockSpec((R, D), lambda i: (i, 0)),
            pl.BlockSpec((R, 1), lambda i: (i, 0)),
        ],
        out_shape=[
            jax.ShapeDtypeStruct((N, D), jnp.float32),
            jax.ShapeDtypeStruct((N, 1), jnp.float32),
        ],
    )(deg, xw)


@functools.partial(
    pl.kernel,
    out_type=jax.ShapeDtypeStruct((NC, NS, NPT, D), jnp.float32),
    mesh=_mesh,
    compiler_params=_CP,
    scratch_types=[
        pltpu.VMEM((NCHUNK, K), jnp.int32),
        pltpu.VMEM((NH, K), jnp.int32),
        pltpu.VMEM((K, D), jnp.float32),
        pltpu.VMEM_SHARED((N, D), jnp.float32),
        pltpu.SemaphoreType.DMA,
    ],
)
def _sc_edges(src_hbm, dst_hbm, y_hbm, agg_out,
              src_v, dst_v, rows_v, aggbuf, sem):
    c = lax.axis_index("c")
    s = lax.axis_index("s")

    def _zero(i, _):
        for t in range(D // 16):
            rows_v[i, pl.ds(t * 16, 16)] = jnp.zeros((16,), jnp.float32)
        return 0

    lax.fori_loop(0, K, _zero, 0)
    nfull = NPT // K
    rem = NPT % K
    for t in range(nfull):
        pltpu.sync_copy(rows_v, aggbuf.at[pl.ds(s * NPT + t * K, K)])
    if rem:
        pltpu.sync_copy(rows_v.at[pl.ds(0, rem)],
                        aggbuf.at[pl.ds(s * NPT + nfull * K, rem)])
    pltpu.sync_copy(src_hbm.at[c, s], src_v)
    pltpu.sync_copy(dst_hbm.at[c, s, pl.ds(0, NH)], dst_v)
    plsc.subcore_barrier()

    for h in range(2):
        if h == 1:
            pltpu.sync_copy(dst_hbm.at[c, s, pl.ds(NH, NH)], dst_v)

        def _accum(j, _):
            pltpu.async_copy(y_hbm.at[src_v.at[h * NH + j]], rows_v,
                             sem).wait()
            pltpu.sync_copy(rows_v, aggbuf.at[dst_v.at[j]], add=True)
            return 0

        lax.fori_loop(0, NH, _accum, 0)
    plsc.subcore_barrier()
    pltpu.sync_copy(aggbuf.at[pl.ds(s * NPT, NPT)], agg_out.at[c, s])


def _tc_final_body(agg_ref, y_ref, dinv_ref, b_ref, o_ref):
    total = agg_ref[0] + agg_ref[1] + y_ref[...]
    o_ref[...] = jnp.maximum(total * dinv_ref[...] + b_ref[...], 0.0)


def _tc_final(agg, y, dinv, b):
    R = 1000
    grid = N // R
    return pl.pallas_call(
        _tc_final_body,
        grid=(grid,),
        in_specs=[
            pl.BlockSpec((NC, R, D), lambda i: (0, i, 0)),
            pl.BlockSpec((R, D), lambda i: (i, 0)),
            pl.BlockSpec((R, 1), lambda i: (i, 0)),
            pl.BlockSpec((1, D), lambda i: (0, 0)),
        ],
        out_specs=pl.BlockSpec((R, D), lambda i: (i, 0)),
        out_shape=jax.ShapeDtypeStruct((N, D), jnp.float32),
    )(agg, y, dinv, b)


@jax.jit
def kernel(x, edge_index, W, b):
    src = edge_index[0].reshape(NC, NS, NCHUNK, K)
    dst = edge_index[1].reshape(NC, NS, NCHUNK, K)
    dstd = edge_index[1].reshape(NC, NS, NCHUNKD, KD)
    ones_rows = jnp.ones((KD, 16), jnp.float32)
    xw = _tc_matmul(x, W)
    deg = _sc_degree(dstd, ones_rows).reshape(NC, N, 16)
    y, dinv = _tc_scale(deg, xw)
    agg = _sc_edges(src, dst, y).reshape(NC, N, D)
    return _tc_final(agg, y, dinv, b.reshape(1, D))


# speedup vs baseline: 1.0123x; 1.0030x over previous
import functools

import jax
import jax.numpy as jnp
from jax import lax
from jax.experimental import pallas as pl
from jax.experimental.pallas import tpu as pltpu
from jax.experimental.pallas import tpu_sc as plsc

N = 10000
E = 320000
D = 128
NC = 2
NS = 16
NW = NC * NS
EPT = E // NW
KD = 250
NCHUNKD = EPT // KD
K = 250
NCHUNK = EPT // K
NH = NCHUNK // 2
NPT = N // NS

_mesh = plsc.VectorSubcoreMesh(core_axis_name="c", subcore_axis_name="s")
_CP = pltpu.CompilerParams(use_tc_tiling_on_sc=False)


@functools.partial(
    pl.kernel,
    out_type=jax.ShapeDtypeStruct((NC, NS, NPT, 16), jnp.float32),
    mesh=_mesh,
    compiler_params=_CP,
    scratch_types=[
        pltpu.VMEM((NCHUNKD, KD), jnp.int32),
        pltpu.VMEM((KD, 16), jnp.float32),
        pltpu.VMEM((NPT, 16), jnp.float32),
        pltpu.VMEM_SHARED((N, 16), jnp.float32),
    ],
)
def _sc_degree(dst_hbm, ones_hbm, deg_out, dst_v, ones_v, zbuf, degbuf):
    c = lax.axis_index("c")
    s = lax.axis_index("s")

    def _zero(i, _):
        zbuf[i] = jnp.zeros((16,), jnp.float32)
        return 0

    lax.fori_loop(0, NPT, _zero, 0)
    pltpu.sync_copy(zbuf, degbuf.at[pl.ds(s * NPT, NPT)])
    pltpu.sync_copy(dst_hbm.at[c, s], dst_v)
    pltpu.sync_copy(ones_hbm, ones_v)
    plsc.subcore_barrier()

    def _accum(j, _):
        pltpu.sync_copy(ones_v, degbuf.at[dst_v.at[j]], add=True)
        return 0

    lax.fori_loop(0, NCHUNKD, _accum, 0)
    plsc.subcore_barrier()
    pltpu.sync_copy(degbuf.at[pl.ds(s * NPT, NPT)], deg_out.at[c, s])


def _tc_matmul_body(x_ref, w_ref, xw_ref):
    xw_ref[...] = jnp.dot(x_ref[...], w_ref[...],
                          preferred_element_type=jnp.float32)


def _tc_matmul(x, W):
    R = 2000
    return pl.pallas_call(
        _tc_matmul_body,
        grid=(N // R,),
        in_specs=[
            pl.BlockSpec((R, D), lambda i: (i, 0)),
            pl.BlockSpec((D, D), lambda i: (0, 0)),
        ],
        out_specs=pl.BlockSpec((R, D), lambda i: (i, 0)),
        out_shape=jax.ShapeDtypeStruct((N, D), jnp.float32),
    )(x, W)


def _tc_scale_body(deg_ref, xw_ref, y_ref, dinv_ref):
    deg = deg_ref[0, :, 0] + deg_ref[1, :, 0] + 1.0
    dinv = lax.rsqrt(deg)
    y_ref[...] = xw_ref[...] * dinv[:, None]
    dinv_ref[...] = dinv[:, None]


def _tc_scale(deg, xw):
    R = 2000
    grid = N // R
    return pl.pallas_call(
        _tc_scale_body,
        grid=(grid,),
        in_specs=[
            pl.BlockSpec((NC, R, 16), lambda i: (0, i, 0)),
            pl.BlockSpec((R, D), lambda i: (i, 0)),
        ],
        out_specs=[
            pl.BlockSpec((R, D), lambda i: (i, 0)),
            pl.BlockSpec((R, 1), lambda i: (i, 0)),
        ],
        out_shape=[
            jax.ShapeDtypeStruct((N, D), jnp.float32),
            jax.ShapeDtypeStruct((N, 1), jnp.float32),
        ],
    )(deg, xw)


@functools.partial(
    pl.kernel,
    out_type=jax.ShapeDtypeStruct((NC, NS, NPT, D), jnp.float32),
    mesh=_mesh,
    compiler_params=_CP,
    scratch_types=[
        pltpu.VMEM((NCHUNK, K), jnp.int32),
        pltpu.VMEM((NH, K), jnp.int32),
        pltpu.VMEM((K, D), jnp.float32),
        pltpu.VMEM_SHARED((N, D), jnp.float32),
        pltpu.SemaphoreType.DMA,
    ],
)
def _sc_edges(src_hbm, dst_hbm, y_hbm, agg_out,
              src_v, dst_v, rows_v, aggbuf, sem):
    c = lax.axis_index("c")
    s = lax.axis_index("s")

    def _zero(i, _):
        for t in range(D // 16):
            rows_v[i, pl.ds(t * 16, 16)] = jnp.zeros((16,), jnp.float32)
        return 0

    lax.fori_loop(0, K, _zero, 0)
    nfull = NPT // K
    rem = NPT % K
    for t in range(nfull):
        pltpu.sync_copy(rows_v, aggbuf.at[pl.ds(s * NPT + t * K, K)])
    if rem:
        pltpu.sync_copy(rows_v.at[pl.ds(0, rem)],
                        aggbuf.at[pl.ds(s * NPT + nfull * K, rem)])
    pltpu.sync_copy(src_hbm.at[c, s], src_v)
    pltpu.sync_copy(dst_hbm.at[c, s, pl.ds(0, NH)], dst_v)
    plsc.subcore_barrier()

    for h in range(2):
        if h == 1:
            pltpu.sync_copy(dst_hbm.at[c, s, pl.ds(NH, NH)], dst_v)

        def _accum(j, _):
            pltpu.async_copy(y_hbm.at[src_v.at[h * NH + j]], rows_v,
                             sem).wait()
            pltpu.sync_copy(rows_v, aggbuf.at[dst_v.at[j]], add=True)
            return 0

        lax.fori_loop(0, NH, _accum, 0)
    plsc.subcore_barrier()
    pltpu.sync_copy(aggbuf.at[pl.ds(s * NPT, NPT)], agg_out.at[c, s])


def _tc_final_body(agg_ref, y_ref, dinv_ref, b_ref, o_ref):
    total = agg_ref[0] + agg_ref[1] + y_ref[...]
    o_ref[...] = jnp.maximum(total * dinv_ref[...] + b_ref[...], 0.0)


def _tc_final(agg, y, dinv, b):
    R = 2000
    grid = N // R
    return pl.pallas_call(
        _tc_final_body,
        grid=(grid,),
        in_specs=[
            pl.BlockSpec((NC, R, D), lambda i: (0, i, 0)),
            pl.BlockSpec((R, D), lambda i: (i, 0)),
            pl.BlockSpec((R, 1), lambda i: (i, 0)),
            pl.BlockSpec((1, D), lambda i: (0, 0)),
        ],
        out_specs=pl.BlockSpec((R, D), lambda i: (i, 0)),
        out_shape=jax.ShapeDtypeStruct((N, D), jnp.float32),
    )(agg, y, dinv, b)


@jax.jit
def kernel(x, edge_index, W, b):
    src = edge_index[0].reshape(NC, NS, NCHUNK, K)
    dst = edge_index[1].reshape(NC, NS, NCHUNK, K)
    dstd = edge_index[1].reshape(NC, NS, NCHUNKD, KD)
    ones_rows = jnp.ones((KD, 16), jnp.float32)
    xw = _tc_matmul(x, W)
    deg = _sc_degree(dstd, ones_rows).reshape(NC, N, 16)
    y, dinv = _tc_scale(deg, xw)
    agg = _sc_edges(src, dst, y).reshape(NC, N, D)
    return _tc_final(agg, y, dinv, b.reshape(1, D))


# speedup vs baseline: 1.0190x; 1.0066x over previous
import functools

import jax
import jax.numpy as jnp
from jax import lax
from jax.experimental import pallas as pl
from jax.experimental.pallas import tpu as pltpu
from jax.experimental.pallas import tpu_sc as plsc

N = 10000
E = 320000
D = 128
NC = 2
NS = 16
NW = NC * NS
EPT = E // NW
KD = 250
NCHUNKD = EPT // KD
K = 250
NCHUNK = EPT // K
NH = NCHUNK // 2
NPT = N // NS

_mesh = plsc.VectorSubcoreMesh(core_axis_name="c", subcore_axis_name="s")
_CP = pltpu.CompilerParams(use_tc_tiling_on_sc=False)


@functools.partial(
    pl.kernel,
    out_type=jax.ShapeDtypeStruct((NC, NS, NPT, 16), jnp.float32),
    mesh=_mesh,
    compiler_params=_CP,
    scratch_types=[
        pltpu.VMEM((NCHUNKD, KD), jnp.int32),
        pltpu.VMEM((KD, 16), jnp.float32),
        pltpu.VMEM((NPT, 16), jnp.float32),
        pltpu.VMEM_SHARED((N, 16), jnp.float32),
    ],
)
def _sc_degree(dst_hbm, ones_hbm, deg_out, dst_v, ones_v, zbuf, degbuf):
    c = lax.axis_index("c")
    s = lax.axis_index("s")

    def _zero(i, _):
        zbuf[i] = jnp.zeros((16,), jnp.float32)
        return 0

    lax.fori_loop(0, NPT, _zero, 0)
    pltpu.sync_copy(zbuf, degbuf.at[pl.ds(s * NPT, NPT)])
    pltpu.sync_copy(dst_hbm.at[c, s], dst_v)
    pltpu.sync_copy(ones_hbm, ones_v)
    plsc.subcore_barrier()

    def _accum(j, _):
        pltpu.sync_copy(ones_v, degbuf.at[dst_v.at[j]], add=True)
        return 0

    lax.fori_loop(0, NCHUNKD, _accum, 0)
    plsc.subcore_barrier()
    pltpu.sync_copy(degbuf.at[pl.ds(s * NPT, NPT)], deg_out.at[c, s])


def _tc_matmul_body(x_ref, w_ref, xw_ref):
    xw_ref[...] = jnp.dot(x_ref[...], w_ref[...],
                          preferred_element_type=jnp.float32)


def _tc_matmul(x, W):
    R = 5000
    return pl.pallas_call(
        _tc_matmul_body,
        grid=(N // R,),
        in_specs=[
            pl.BlockSpec((R, D), lambda i: (i, 0)),
            pl.BlockSpec((D, D), lambda i: (0, 0)),
        ],
        out_specs=pl.BlockSpec((R, D), lambda i: (i, 0)),
        out_shape=jax.ShapeDtypeStruct((N, D), jnp.float32),
    )(x, W)


def _tc_scale_body(deg_ref, xw_ref, y_ref, dinv_ref):
    deg = deg_ref[0, :, 0] + deg_ref[1, :, 0] + 1.0
    dinv = lax.rsqrt(deg)
    y_ref[...] = xw_ref[...] * dinv[:, None]
    dinv_ref[...] = dinv[:, None]


def _tc_scale(deg, xw):
    R = 5000
    grid = N // R
    return pl.pallas_call(
        _tc_scale_body,
        grid=(grid,),
        in_specs=[
            pl.BlockSpec((NC, R, 16), lambda i: (0, i, 0)),
            pl.BlockSpec((R, D), lambda i: (i, 0)),
        ],
        out_specs=[
            pl.BlockSpec((R, D), lambda i: (i, 0)),
            pl.BlockSpec((R, 1), lambda i: (i, 0)),
        ],
        out_shape=[
            jax.ShapeDtypeStruct((N, D), jnp.float32),
            jax.ShapeDtypeStruct((N, 1), jnp.float32),
        ],
    )(deg, xw)


@functools.partial(
    pl.kernel,
    out_type=jax.ShapeDtypeStruct((NC, NS, NPT, D), jnp.float32),
    mesh=_mesh,
    compiler_params=_CP,
    scratch_types=[
        pltpu.VMEM((NCHUNK, K), jnp.int32),
        pltpu.VMEM((NH, K), jnp.int32),
        pltpu.VMEM((K, D), jnp.float32),
        pltpu.VMEM_SHARED((N, D), jnp.float32),
        pltpu.SemaphoreType.DMA,
    ],
)
def _sc_edges(src_hbm, dst_hbm, y_hbm, agg_out,
              src_v, dst_v, rows_v, aggbuf, sem):
    c = lax.axis_index("c")
    s = lax.axis_index("s")

    def _zero(i, _):
        for t in range(D // 16):
            rows_v[i, pl.ds(t * 16, 16)] = jnp.zeros((16,), jnp.float32)
        return 0

    lax.fori_loop(0, K, _zero, 0)
    nfull = NPT // K
    rem = NPT % K
    for t in range(nfull):
        pltpu.sync_copy(rows_v, aggbuf.at[pl.ds(s * NPT + t * K, K)])
    if rem:
        pltpu.sync_copy(rows_v.at[pl.ds(0, rem)],
                        aggbuf.at[pl.ds(s * NPT + nfull * K, rem)])
    pltpu.sync_copy(src_hbm.at[c, s], src_v)
    pltpu.sync_copy(dst_hbm.at[c, s, pl.ds(0, NH)], dst_v)
    plsc.subcore_barrier()

    for h in range(2):
        if h == 1:
            pltpu.sync_copy(dst_hbm.at[c, s, pl.ds(NH, NH)], dst_v)

        def _accum(j, _):
            pltpu.async_copy(y_hbm.at[src_v.at[h * NH + j]], rows_v,
                             sem).wait()
            pltpu.sync_copy(rows_v, aggbuf.at[dst_v.at[j]], add=True)
            return 0

        lax.fori_loop(0, NH, _accum, 0)
    plsc.subcore_barrier()
    pltpu.sync_copy(aggbuf.at[pl.ds(s * NPT, NPT)], agg_out.at[c, s])


def _tc_final_body(agg_ref, y_ref, dinv_ref, b_ref, o_ref):
    total = agg_ref[0] + agg_ref[1] + y_ref[...]
    o_ref[...] = jnp.maximum(total * dinv_ref[...] + b_ref[...], 0.0)


def _tc_final(agg, y, dinv, b):
    R = 5000
    grid = N // R
    return pl.pallas_call(
        _tc_final_body,
        grid=(grid,),
        in_specs=[
            pl.BlockSpec((NC, R, D), lambda i: (0, i, 0)),
            pl.BlockSpec((R, D), lambda i: (i, 0)),
            pl.BlockSpec((R, 1), lambda i: (i, 0)),
            pl.BlockSpec((1, D), lambda i: (0, 0)),
        ],
        out_specs=pl.BlockSpec((R, D), lambda i: (i, 0)),
        out_shape=jax.ShapeDtypeStruct((N, D), jnp.float32),
    )(agg, y, dinv, b)


@jax.jit
def kernel(x, edge_index, W, b):
    src = edge_index[0].reshape(NC, NS, NCHUNK, K)
    dst = edge_index[1].reshape(NC, NS, NCHUNK, K)
    dstd = edge_index[1].reshape(NC, NS, NCHUNKD, KD)
    ones_rows = jnp.ones((KD, 16), jnp.float32)
    xw = _tc_matmul(x, W)
    deg = _sc_degree(dstd, ones_rows).reshape(NC, N, 16)
    y, dinv = _tc_scale(deg, xw)
    agg = _sc_edges(src, dst, y).reshape(NC, N, D)
    return _tc_final(agg, y, dinv, b.reshape(1, D))
